# baseline (speedup 1.0000x reference)
import functools

import jax
import jax.numpy as jnp
from jax.experimental import pallas as pl
from jax.experimental.pallas import tpu as pltpu

_TT = 16


def _tc_kernel(eids_ref, regions_ref, spikes_ref, w_ref, b_ref, slot_ref,
               regemb_ref, eidemb_ref, out_ref, base_ref):
    b_idx = pl.program_id(0)
    t_idx = pl.program_id(1)
    n = slot_ref.shape[0]

    @pl.when(t_idx == 0)
    def _build_base():
        regions = regions_ref[0, :, :]
        nregions = regemb_ref.shape[0]
        oh = (regions == jax.lax.broadcasted_iota(
            jnp.int32, (n, nregions), 1)).astype(jnp.float32)
        reg = jnp.dot(oh, regemb_ref[...], preferred_element_type=jnp.float32)

        e = eids_ref[b_idx]
        neids = eidemb_ref.shape[0]
        ohe = (jax.lax.broadcasted_iota(jnp.int32, (8, neids), 1) == e
               ).astype(jnp.float32)
        ev = jnp.dot(ohe, eidemb_ref[...], preferred_element_type=jnp.float32)

        base_ref[...] = slot_ref[...] + reg + ev[0:1, :] + b_ref[...]

    spikes = spikes_ref[0]
    out_ref[0] = (spikes[:, :, None] * w_ref[0, :][None, None, :]
                  + base_ref[...][None, :, :])


@jax.jit
def kernel(spikes, neuron_regions, eids, w_spike, b_spike, neuron_slot,
           region_emb, eid_emb):
    B, T, N = spikes.shape
    D = neuron_slot.shape[1]
    regions3 = neuron_regions.astype(jnp.int32).reshape(B, N, 1)
    eids32 = eids.astype(jnp.int32)
    w2 = w_spike.reshape(1, D)
    b2 = b_spike.reshape(1, D)

    grid = (B, T // _TT)
    return pl.pallas_call(
        _tc_kernel,
        grid=grid,
        in_specs=[
            pl.BlockSpec(memory_space=pltpu.SMEM),
            pl.BlockSpec((1, N, 1), lambda b, t: (b, 0, 0)),
            pl.BlockSpec((1, _TT, N), lambda b, t: (b, t, 0)),
            pl.BlockSpec((1, D), lambda b, t: (0, 0)),
            pl.BlockSpec((1, D), lambda b, t: (0, 0)),
            pl.BlockSpec((N, D), lambda b, t: (0, 0)),
            pl.BlockSpec(region_emb.shape, lambda b, t: (0, 0)),
            pl.BlockSpec(eid_emb.shape, lambda b, t: (0, 0)),
        ],
        out_specs=pl.BlockSpec((1, _TT, N, D), lambda b, t: (b, t, 0, 0)),
        out_shape=jax.ShapeDtypeStruct((B, T, N, D), jnp.float32),
        scratch_shapes=[pltpu.VMEM((N, D), jnp.float32)],
    )(eids32, regions3, spikes, w2, b2, neuron_slot, region_emb, eid_emb)
